# baseline (speedup 1.0000x reference)
import functools

import jax
import jax.numpy as jnp
from jax import lax
from jax.experimental import pallas as pl
from jax.experimental.pallas import tpu as pltpu
from jax.experimental.pallas import tpu_sc as plsc

_VOCAB = 1000000
_EMBD = 32
_BATCH = 16384
_NNEG = 5

_NC = 2
_NS = 16
_L = 16
_NW = _NC * _NS
_BPW = _BATCH // _NW
_G = _BPW // _L
_CH = 128
_UCH = _BPW // _CH
_NCH = _BPW * _NNEG // _CH


def _sc_scores_body(u_idx_hbm, p_idx_hbm, n_idx_hbm, U_hbm, V_hbm,
                    pos_out, neg_out,
                    uidx, pidx, nidx, urows, prows, nrows, psc, nsc, sem):
    wid = lax.axis_index("s") * _NC + lax.axis_index("c")
    base = wid * _BPW

    pltpu.sync_copy(u_idx_hbm.at[pl.ds(base, _BPW)], uidx)
    pltpu.sync_copy(p_idx_hbm.at[pl.ds(base, _BPW)], pidx)
    pltpu.sync_copy(n_idx_hbm.at[pl.ds(base * _NNEG, _BPW * _NNEG)], nidx)

    copies = []
    for j in range(_UCH):
        copies.append(pltpu.async_copy(
            U_hbm.at[uidx.at[pl.ds(j * _CH, _CH)]],
            urows.at[pl.ds(j * _CH, _CH)], sem))
        copies.append(pltpu.async_copy(
            V_hbm.at[pidx.at[pl.ds(j * _CH, _CH)]],
            prows.at[pl.ds(j * _CH, _CH)], sem))
    for j in range(_NCH):
        copies.append(pltpu.async_copy(
            V_hbm.at[nidx.at[pl.ds(j * _CH, _CH)]],
            nrows.at[pl.ds(j * _CH, _CH)], sem))
    for c in copies:
        c.wait()

    iot = lax.iota(jnp.int32, _L)

    def group(g, carry):
        rb = g * _L + iot
        nrb = [rb * _NNEG + n for n in range(_NNEG)]
        pos_acc = jnp.zeros((_L,), jnp.float32)
        neg_acc = jnp.zeros((_L,), jnp.float32)
        for d in range(_EMBD):
            dcol = jnp.full((_L,), d, jnp.int32)
            uc = plsc.load_gather(urows, [rb, dcol])
            pc = plsc.load_gather(prows, [rb, dcol])
            nsum = plsc.load_gather(nrows, [nrb[0], dcol])
            for n in range(1, _NNEG):
                nsum = nsum + plsc.load_gather(nrows, [nrb[n], dcol])
            pos_acc = pos_acc + uc * pc
            neg_acc = neg_acc + uc * nsum
        psc[pl.ds(g * _L, _L)] = pos_acc
        nsc[pl.ds(g * _L, _L)] = neg_acc
        return carry

    lax.fori_loop(0, _G, group, 0)

    pltpu.sync_copy(psc, pos_out.at[pl.ds(base, _BPW)])
    pltpu.sync_copy(nsc, neg_out.at[pl.ds(base, _BPW)])


_sc_scores = functools.partial(
    pl.kernel,
    out_type=[jax.ShapeDtypeStruct((_BATCH,), jnp.float32),
              jax.ShapeDtypeStruct((_BATCH,), jnp.float32)],
    mesh=plsc.VectorSubcoreMesh(core_axis_name="c", subcore_axis_name="s"),
    compiler_params=pltpu.CompilerParams(needs_layout_passes=False,
                                         use_tc_tiling_on_sc=False),
    scratch_types=[
        pltpu.VMEM((_BPW,), jnp.int32),
        pltpu.VMEM((_BPW,), jnp.int32),
        pltpu.VMEM((_BPW * _NNEG,), jnp.int32),
        pltpu.VMEM((_BPW, _EMBD), jnp.float32),
        pltpu.VMEM((_BPW, _EMBD), jnp.float32),
        pltpu.VMEM((_BPW * _NNEG, _EMBD), jnp.float32),
        pltpu.VMEM((_BPW,), jnp.float32),
        pltpu.VMEM((_BPW,), jnp.float32),
        pltpu.SemaphoreType.DMA,
    ],
)(_sc_scores_body)


def _tc_final_body(pos_ref, neg_ref, out_ref):
    pos = pos_ref[...]
    neg = neg_ref[...]
    total = (jnp.sum(jax.nn.log_sigmoid(pos))
             + jnp.sum(jax.nn.log_sigmoid(-neg)))
    out_ref[0, 0] = -total / _BATCH


def _tc_final(pos2d, neg2d):
    return pl.pallas_call(
        _tc_final_body,
        out_shape=jax.ShapeDtypeStruct((1, 1), jnp.float32),
        out_specs=pl.BlockSpec(memory_space=pltpu.SMEM),
    )(pos2d, neg2d)


def kernel(u_pos, v_pos, v_neg, U, V):
    u2 = u_pos.astype(jnp.int32)
    p2 = v_pos.astype(jnp.int32)
    n2 = v_neg.astype(jnp.int32).reshape(_BATCH * _NNEG)
    pos, neg = _sc_scores(u2, p2, n2, U, V)
    res = _tc_final(pos.reshape(_BATCH // 128, 128),
                    neg.reshape(_BATCH // 128, 128))
    return res[0, 0]
